# baseline (speedup 1.0000x reference)
import functools

import jax
import jax.numpy as jnp
from jax import lax
from jax.experimental import pallas as pl
from jax.experimental.pallas import tpu as pltpu
from jax.experimental.pallas import tpu_sc as plsc

VOCAB = 50257
HIDDEN = 768
MAX_POS = 2048
BATCH = 4
SEQ = 2048
TOK = BATCH * SEQ
LANES = 16
NC, NS = 2, 16
NW = NC * NS
PPW = SEQ // NW
PQ = 8
CHR = PQ * BATCH
NCHK = PPW // PQ
HV = HIDDEN // LANES

_mesh = plsc.VectorSubcoreMesh(core_axis_name="c", subcore_axis_name="s")


@functools.partial(
    pl.kernel,
    mesh=_mesh,
    out_type=jax.ShapeDtypeStruct((TOK, HIDDEN), jnp.float32),
    scratch_types=[
        pltpu.VMEM((PPW * BATCH,), jnp.int32),
        pltpu.VMEM((PQ, HIDDEN), jnp.float32),
        pltpu.VMEM((PQ, HIDDEN), jnp.float32),
        pltpu.VMEM((CHR, HIDDEN), jnp.float32),
        pltpu.VMEM((CHR, HIDDEN), jnp.float32),
        pltpu.VMEM((BATCH, PQ, HIDDEN), jnp.float32),
        pltpu.VMEM((BATCH, PQ, HIDDEN), jnp.float32),
        pltpu.SemaphoreType.DMA,
        pltpu.SemaphoreType.DMA,
        pltpu.SemaphoreType.DMA,
        pltpu.SemaphoreType.DMA,
        pltpu.SemaphoreType.DMA,
        pltpu.SemaphoreType.DMA,
    ],
)
def _embed(ids_hbm, wte_hbm, wpe_hbm, out_hbm, idx_v, wp0, wp1, rb0, rb1,
           ob0, ob1, ws0, ws1, gs0, gs1, os0, os1):
    wid = lax.axis_index("s") * NC + lax.axis_index("c")
    pos0 = wid * PPW
    wpe_r = [wp0, wp1]
    rows_r = [rb0, rb1]
    out_r = [ob0, ob1]
    wsems = [ws0, ws1]
    gsems = [gs0, gs1]
    osems = [os0, os1]

    pltpu.sync_copy(ids_hbm.at[wid], idx_v)

    def in_copies(k, buf):
        g = pltpu.make_async_copy(
            wte_hbm.at[idx_v.at[pl.ds(k * CHR, CHR)]], rows_r[buf],
            gsems[buf])
        w = pltpu.make_async_copy(
            wpe_hbm.at[pl.ds(pos0 + k * PQ, PQ)], wpe_r[buf], wsems[buf])
        return [g, w]

    def out_copies(k, buf):
        return [
            pltpu.make_async_copy(
                out_r[buf].at[b],
                out_hbm.at[pl.ds(b * SEQ + pos0 + k * PQ, PQ)], osems[buf])
            for b in range(BATCH)
        ]

    for d in in_copies(0, 0) + in_copies(1, 1):
        d.start()

    def chunk_pair(t, carry):
        for sub in range(2):
            k = 2 * t + sub
            buf = sub

            @pl.when(t > 0)
            def _drain_out():
                for d in out_copies(k - 2, buf):
                    d.wait()

            for d in in_copies(k, buf):
                d.wait()

            @plsc.parallel_loop(0, PQ * 4, 1, unroll=1)
            def _add(i):
                p = i // 4
                q = i - p * 4
                sls = [pl.ds((q * 12 + j) * LANES, LANES) for j in range(12)]
                wvs = [wpe_r[buf][p, s] for s in sls]
                for b in range(BATCH):
                    for j, s in enumerate(sls):
                        out_r[buf][b, p, s] = (
                            rows_r[buf][p * BATCH + b, s] + wvs[j])

            for d in out_copies(k, buf):
                d.start()

            @pl.when(k + 2 < NCHK)
            def _issue_next():
                for d in in_copies(k + 2, buf):
                    d.start()

        return carry

    lax.fori_loop(0, NCHK // 2, chunk_pair, 0)
    for d in out_copies(NCHK - 2, 0) + out_copies(NCHK - 1, 1):
        d.wait()


def kernel(input_ids, wte, wpe):
    ids = (input_ids.astype(jnp.int32)
           .reshape(BATCH, NW, PPW)
           .transpose(1, 2, 0)
           .reshape(NW, PPW * BATCH))
    out = _embed(ids, wte, wpe)
    return out.reshape(BATCH, SEQ, HIDDEN)


# speedup vs baseline: 1.3435x; 1.3435x over previous
import functools

import jax
import jax.numpy as jnp
from jax import lax
from jax.experimental import pallas as pl
from jax.experimental.pallas import tpu as pltpu
from jax.experimental.pallas import tpu_sc as plsc

VOCAB = 50257
HIDDEN = 768
MAX_POS = 2048
BATCH = 4
SEQ = 2048
TOK = BATCH * SEQ
LANES = 16
NC, NS = 2, 16
NW = NC * NS
TPW = TOK // NW
CH = 16
NCH = TPW // CH
HV = HIDDEN // LANES
NVB = 4
NWP = 3

_mesh = plsc.VectorSubcoreMesh(core_axis_name="c", subcore_axis_name="s")


@functools.partial(
    pl.kernel,
    mesh=_mesh,
    out_type=jax.ShapeDtypeStruct((TOK, HIDDEN), jnp.float32),
    scratch_types=[
        pltpu.VMEM((TPW,), jnp.int32),
        pltpu.VMEM((CH, HIDDEN), jnp.float32),
        pltpu.VMEM((CH, HIDDEN), jnp.float32),
        pltpu.VMEM((CH, HIDDEN), jnp.float32),
        pltpu.VMEM((CH, HIDDEN), jnp.float32),
        pltpu.VMEM((CH, HIDDEN), jnp.float32),
        pltpu.VMEM((CH, HIDDEN), jnp.float32),
        pltpu.VMEM((CH, HIDDEN), jnp.float32),
        pltpu.SemaphoreType.DMA,
        pltpu.SemaphoreType.DMA,
        pltpu.SemaphoreType.DMA,
        pltpu.SemaphoreType.DMA,
        pltpu.SemaphoreType.DMA,
        pltpu.SemaphoreType.DMA,
        pltpu.SemaphoreType.DMA,
        pltpu.SemaphoreType.DMA,
        pltpu.SemaphoreType.DMA,
        pltpu.SemaphoreType.DMA,
        pltpu.SemaphoreType.DMA,
    ],
)
def _embed(ids_hbm, wte_hbm, wpe_hbm, out_hbm, idx_v,
           vb0, vb1, vb2, vb3, wp0, wp1, wp2,
           g0, g1, g2, g3, w0, w1, w2, o0, o1, o2, o3):
    cid = lax.axis_index("c")
    sid = lax.axis_index("s")
    wid = sid * NC + cid
    base = wid * TPW
    pos0 = lax.rem(base, SEQ)
    vbufs = [vb0, vb1, vb2, vb3]
    wbufs = [wp0, wp1, wp2]
    gsems = [g0, g1, g2, g3]
    wsems = [w0, w1, w2]
    osems = [o0, o1, o2, o3]

    pltpu.sync_copy(ids_hbm.at[wid], idx_v)

    def gath(k):
        r = k % NVB
        return pltpu.async_copy(
            wte_hbm.at[idx_v.at[pl.ds(k * CH, CH)]], vbufs[r], gsems[r])

    def wpe(k):
        r = k % NWP
        return pltpu.async_copy(
            wpe_hbm.at[pl.ds(pos0 + k * CH, CH)], wbufs[r], wsems[r])

    def wb(k):
        r = k % NVB
        return pltpu.async_copy(
            vbufs[r], out_hbm.at[pl.ds(base + k * CH, CH)], osems[r])

    dg = {0: gath(0), 1: gath(1)}
    dwp = {0: wpe(0), 1: wpe(1), 2: wpe(2)}
    dw = {}
    for k in range(NCH):
        if k + 2 < NCH:
            if k - 2 >= 0:
                dw[k - 2].wait()
            dg[k + 2] = gath(k + 2)
        dg[k].wait()
        dwp[k].wait()

        rows = vbufs[k % NVB]
        wrows = wbufs[k % NWP]

        def add_row(i, carry):
            for j in range(HV):
                s = pl.ds(j * LANES, LANES)
                plsc.addupdate(rows.at[i, s], wrows[i, s])
            return carry

        lax.fori_loop(0, CH, add_row, 0)
        dw[k] = wb(k)
        if k + 3 < NCH:
            dwp[k + 3] = wpe(k + 3)
    dw[NCH - 4].wait()
    dw[NCH - 3].wait()
    dw[NCH - 2].wait()
    dw[NCH - 1].wait()


def kernel(input_ids, wte, wpe):
    ids = input_ids.reshape(NW, TPW).astype(jnp.int32)
    out = _embed(ids, wte, wpe)
    return out.reshape(BATCH, SEQ, HIDDEN)


# speedup vs baseline: 1.4995x; 1.1161x over previous
import functools

import jax
import jax.numpy as jnp
from jax import lax
from jax.experimental import pallas as pl
from jax.experimental.pallas import tpu as pltpu
from jax.experimental.pallas import tpu_sc as plsc

VOCAB = 50257
HIDDEN = 768
MAX_POS = 2048
BATCH = 4
SEQ = 2048
TOK = BATCH * SEQ
LANES = 16
NC, NS = 2, 16
NW = NC * NS
TPW = TOK // NW
PPW = SEQ // NW
CH = 16
NPC = PPW // CH
NCH = NPC * BATCH
HV = HIDDEN // LANES
NVB = 4
NWP = 2

_mesh = plsc.VectorSubcoreMesh(core_axis_name="c", subcore_axis_name="s")


@functools.partial(
    pl.kernel,
    mesh=_mesh,
    out_type=jax.ShapeDtypeStruct((TOK, HIDDEN), jnp.float32),
    scratch_types=[
        pltpu.VMEM((TPW,), jnp.int32),
        pltpu.VMEM((CH, HIDDEN), jnp.float32),
        pltpu.VMEM((CH, HIDDEN), jnp.float32),
        pltpu.VMEM((CH, HIDDEN), jnp.float32),
        pltpu.VMEM((CH, HIDDEN), jnp.float32),
        pltpu.VMEM((CH, HIDDEN), jnp.float32),
        pltpu.VMEM((CH, HIDDEN), jnp.float32),
        pltpu.SemaphoreType.DMA,
        pltpu.SemaphoreType.DMA,
        pltpu.SemaphoreType.DMA,
        pltpu.SemaphoreType.DMA,
        pltpu.SemaphoreType.DMA,
        pltpu.SemaphoreType.DMA,
        pltpu.SemaphoreType.DMA,
        pltpu.SemaphoreType.DMA,
        pltpu.SemaphoreType.DMA,
        pltpu.SemaphoreType.DMA,
    ],
)
def _embed(ids_hbm, wte_hbm, wpe_hbm, out_hbm, idx_v,
           vb0, vb1, vb2, vb3, wp0, wp1,
           g0, g1, g2, g3, w0, w1, o0, o1, o2, o3):
    cid = lax.axis_index("c")
    sid = lax.axis_index("s")
    wid = sid * NC + cid
    pos0 = wid * PPW
    vbufs = [vb0, vb1, vb2, vb3]
    wbufs = [wp0, wp1]
    gsems = [g0, g1, g2, g3]
    wsems = [w0, w1]
    osems = [o0, o1, o2, o3]

    pltpu.sync_copy(ids_hbm.at[wid], idx_v)

    def gath(k):
        r = k % NVB
        return pltpu.async_copy(
            wte_hbm.at[idx_v.at[pl.ds(k * CH, CH)]], vbufs[r], gsems[r])

    def wpe(q):
        r = q % NWP
        return pltpu.async_copy(
            wpe_hbm.at[pl.ds(pos0 + q * CH, CH)], wbufs[r], wsems[r])

    def wb(k):
        q, b = divmod(k, BATCH)
        r = k % NVB
        return pltpu.async_copy(
            vbufs[r], out_hbm.at[pl.ds(b * SEQ + pos0 + q * CH, CH)],
            osems[r])

    dg = {0: gath(0), 1: gath(1)}
    dwp = {0: wpe(0), 1: wpe(1)}
    dw = {}
    for k in range(NCH):
        q, b = divmod(k, BATCH)
        if k + 2 < NCH:
            if k - 2 >= 0:
                dw[k - 2].wait()
            dg[k + 2] = gath(k + 2)
        if b == 0:
            dwp[q].wait()
        dg[k].wait()

        rows = vbufs[k % NVB]
        wrows = wbufs[q % NWP]

        def add_row(i, carry):
            for j in range(HV):
                s = pl.ds(j * LANES, LANES)
                plsc.addupdate(rows.at[i, s], wrows[i, s])
            return carry

        lax.fori_loop(0, CH, add_row, 0)
        dw[k] = wb(k)
        if b == BATCH - 1 and q + 2 < NPC:
            dwp[q + 2] = wpe(q + 2)
    dw[NCH - 4].wait()
    dw[NCH - 3].wait()
    dw[NCH - 2].wait()
    dw[NCH - 1].wait()


def kernel(input_ids, wte, wpe):
    ids = (input_ids.astype(jnp.int32)
           .reshape(BATCH, NW, NPC, CH)
           .transpose(1, 2, 0, 3)
           .reshape(NW, TPW))
    out = _embed(ids, wte, wpe)
    return out.reshape(BATCH, SEQ, HIDDEN)


# speedup vs baseline: 1.5446x; 1.0301x over previous
import functools

import jax
import jax.numpy as jnp
from jax import lax
from jax.experimental import pallas as pl
from jax.experimental.pallas import tpu as pltpu
from jax.experimental.pallas import tpu_sc as plsc

VOCAB = 50257
HIDDEN = 768
MAX_POS = 2048
BATCH = 4
SEQ = 2048
TOK = BATCH * SEQ
LANES = 16
NC, NS = 2, 16
NW = NC * NS
TPW = TOK // NW
PPW = SEQ // NW
CH = 16
NPC = PPW // CH
NCH = NPC * BATCH
HV = HIDDEN // LANES
NVB = 6
NWP = 2

_mesh = plsc.VectorSubcoreMesh(core_axis_name="c", subcore_axis_name="s")


@functools.partial(
    pl.kernel,
    mesh=_mesh,
    out_type=jax.ShapeDtypeStruct((TOK, HIDDEN), jnp.float32),
    scratch_types=[
        pltpu.VMEM((TPW,), jnp.int32),
        pltpu.VMEM((CH, HIDDEN), jnp.float32),
        pltpu.VMEM((CH, HIDDEN), jnp.float32),
        pltpu.VMEM((CH, HIDDEN), jnp.float32),
        pltpu.VMEM((CH, HIDDEN), jnp.float32),
        pltpu.VMEM((CH, HIDDEN), jnp.float32),
        pltpu.VMEM((CH, HIDDEN), jnp.float32),
        pltpu.VMEM((CH, HIDDEN), jnp.float32),
        pltpu.VMEM((CH, HIDDEN), jnp.float32),
        pltpu.SemaphoreType.DMA,
        pltpu.SemaphoreType.DMA,
        pltpu.SemaphoreType.DMA,
        pltpu.SemaphoreType.DMA,
        pltpu.SemaphoreType.DMA,
        pltpu.SemaphoreType.DMA,
        pltpu.SemaphoreType.DMA,
        pltpu.SemaphoreType.DMA,
        pltpu.SemaphoreType.DMA,
        pltpu.SemaphoreType.DMA,
        pltpu.SemaphoreType.DMA,
        pltpu.SemaphoreType.DMA,
        pltpu.SemaphoreType.DMA,
        pltpu.SemaphoreType.DMA,
    ],
)
def _embed(ids_hbm, wte_hbm, wpe_hbm, out_hbm, idx_v,
           vb0, vb1, vb2, vb3, vb4, vb5, wp0, wp1,
           g0, g1, g2, g3, g4, g5, w0, w1, o0, o1, o2, o3, o4, o5):
    cid = lax.axis_index("c")
    sid = lax.axis_index("s")
    wid = sid * NC + cid
    pos0 = wid * PPW
    vbufs = [vb0, vb1, vb2, vb3, vb4, vb5]
    wbufs = [wp0, wp1]
    gsems = [g0, g1, g2, g3, g4, g5]
    wsems = [w0, w1]
    osems = [o0, o1, o2, o3, o4, o5]

    pltpu.sync_copy(ids_hbm.at[wid], idx_v)

    def gath(k):
        r = k % NVB
        return pltpu.async_copy(
            wte_hbm.at[idx_v.at[pl.ds(k * CH, CH)]], vbufs[r], gsems[r])

    def wpe(q):
        r = q % NWP
        return pltpu.async_copy(
            wpe_hbm.at[pl.ds(pos0 + q * CH, CH)], wbufs[r], wsems[r])

    def wb(k):
        q, b = divmod(k, BATCH)
        r = k % NVB
        return pltpu.async_copy(
            vbufs[r], out_hbm.at[pl.ds(b * SEQ + pos0 + q * CH, CH)],
            osems[r])

    dg = {j: gath(j) for j in range(4)}
    dwp = {0: wpe(0), 1: wpe(1)}
    dw = {}
    for k in range(NCH):
        q, b = divmod(k, BATCH)
        if k + 4 < NCH:
            if k - 2 >= 0:
                dw[k - 2].wait()
            dg[k + 4] = gath(k + 4)
        if b == 0:
            dwp[q].wait()
        dg[k].wait()

        rows = vbufs[k % NVB]
        wrows = wbufs[q % NWP]

        def add_row(i, carry):
            for j in range(HV):
                s = pl.ds(j * LANES, LANES)
                plsc.addupdate(rows.at[i, s], wrows[i, s])
            return carry

        lax.fori_loop(0, CH, add_row, 0)
        dw[k] = wb(k)
        if b == BATCH - 1 and q + 2 < NPC:
            dwp[q + 2] = wpe(q + 2)
    for j in range(NCH - 6, NCH):
        dw[j].wait()


def kernel(input_ids, wte, wpe):
    ids = (input_ids.astype(jnp.int32)
           .reshape(BATCH, NW, NPC, CH)
           .transpose(1, 2, 0, 3)
           .reshape(NW, TPW))
    out = _embed(ids, wte, wpe)
    return out.reshape(BATCH, SEQ, HIDDEN)


# speedup vs baseline: 1.5476x; 1.0019x over previous
import functools

import jax
import jax.numpy as jnp
from jax import lax
from jax.experimental import pallas as pl
from jax.experimental.pallas import tpu as pltpu
from jax.experimental.pallas import tpu_sc as plsc

VOCAB = 50257
HIDDEN = 768
MAX_POS = 2048
BATCH = 4
SEQ = 2048
TOK = BATCH * SEQ
LANES = 16
NC, NS = 2, 16
NW = NC * NS
TPW = TOK // NW
PPW = SEQ // NW
CH = 16
NPC = PPW // CH
NCH = NPC * BATCH
HV = HIDDEN // LANES
NVB = 6
NWP = 2

_mesh = plsc.VectorSubcoreMesh(core_axis_name="c", subcore_axis_name="s")


@functools.partial(
    pl.kernel,
    mesh=_mesh,
    out_type=jax.ShapeDtypeStruct((TOK, HIDDEN), jnp.float32),
    scratch_types=[
        pltpu.VMEM((TPW,), jnp.int32),
        pltpu.VMEM((CH, HIDDEN), jnp.float32),
        pltpu.VMEM((CH, HIDDEN), jnp.float32),
        pltpu.VMEM((CH, HIDDEN), jnp.float32),
        pltpu.VMEM((CH, HIDDEN), jnp.float32),
        pltpu.VMEM((CH, HIDDEN), jnp.float32),
        pltpu.VMEM((CH, HIDDEN), jnp.float32),
        pltpu.VMEM((CH, HIDDEN), jnp.float32),
        pltpu.VMEM((CH, HIDDEN), jnp.float32),
        pltpu.SemaphoreType.DMA,
        pltpu.SemaphoreType.DMA,
        pltpu.SemaphoreType.DMA,
        pltpu.SemaphoreType.DMA,
        pltpu.SemaphoreType.DMA,
        pltpu.SemaphoreType.DMA,
        pltpu.SemaphoreType.DMA,
        pltpu.SemaphoreType.DMA,
        pltpu.SemaphoreType.DMA,
        pltpu.SemaphoreType.DMA,
        pltpu.SemaphoreType.DMA,
        pltpu.SemaphoreType.DMA,
        pltpu.SemaphoreType.DMA,
        pltpu.SemaphoreType.DMA,
        pltpu.SemaphoreType.DMA,
    ],
)
def _embed(ids_hbm, wte_hbm, wpe_hbm, out_hbm, idx_v,
           vb0, vb1, vb2, vb3, vb4, vb5, wp0, wp1,
           g0, g1, g2, g3, g4, g5, w0, w1, o0, o1, o2, o3, o4, o5, isem):
    cid = lax.axis_index("c")
    sid = lax.axis_index("s")
    wid = sid * NC + cid
    pos0 = wid * PPW
    vbufs = [vb0, vb1, vb2, vb3, vb4, vb5]
    wbufs = [wp0, wp1]
    gsems = [g0, g1, g2, g3, g4, g5]
    wsems = [w0, w1]
    osems = [o0, o1, o2, o3, o4, o5]

    did = [
        pltpu.async_copy(
            ids_hbm.at[b, pl.ds(pos0 + q * CH, CH)],
            idx_v.at[pl.ds((q * BATCH + b) * CH, CH)], isem)
        for q in range(NPC) for b in range(BATCH)
    ]
    for d in did:
        d.wait()

    def gath(k):
        r = k % NVB
        return pltpu.async_copy(
            wte_hbm.at[idx_v.at[pl.ds(k * CH, CH)]], vbufs[r], gsems[r])

    def wpe(q):
        r = q % NWP
        return pltpu.async_copy(
            wpe_hbm.at[pl.ds(pos0 + q * CH, CH)], wbufs[r], wsems[r])

    def wb(k):
        q, b = divmod(k, BATCH)
        r = k % NVB
        return pltpu.async_copy(
            vbufs[r], out_hbm.at[pl.ds(b * SEQ + pos0 + q * CH, CH)],
            osems[r])

    dg = {j: gath(j) for j in range(4)}
    dwp = {0: wpe(0), 1: wpe(1)}
    dw = {}
    for k in range(NCH):
        q, b = divmod(k, BATCH)
        if k + 4 < NCH:
            if k - 2 >= 0:
                dw[k - 2].wait()
            dg[k + 4] = gath(k + 4)
        if b == 0:
            dwp[q].wait()
        dg[k].wait()

        rows = vbufs[k % NVB]
        wrows = wbufs[q % NWP]

        def add_row(i, carry):
            for j in range(HV):
                s = pl.ds(j * LANES, LANES)
                plsc.addupdate(rows.at[i, s], wrows[i, s])
            return carry

        lax.fori_loop(0, CH, add_row, 0)
        dw[k] = wb(k)
        if b == BATCH - 1 and q + 2 < NPC:
            dwp[q + 2] = wpe(q + 2)
    for j in range(NCH - 6, NCH):
        dw[j].wait()


def kernel(input_ids, wte, wpe):
    ids = input_ids.astype(jnp.int32)
    out = _embed(ids, wte, wpe)
    return out.reshape(BATCH, SEQ, HIDDEN)
